# baseline (speedup 1.0000x reference)
import jax
import jax.numpy as jnp
import numpy as np
from jax import lax
from jax.experimental import pallas as pl
from jax.experimental.pallas import tpu as pltpu
from jax.experimental.pallas import tpu_sc as plsc

_B, _A, _GH, _GW = 16, 3, 64, 64
_C = 80
_NCH = 5 + _C
_NT = 64
_NCELL = _B * _A * _GH * _GW
_INPUT_DIM = 512.0
_NO_OBJ_W = 0.5
_CLS_W = 1.0
_SA = np.array([[80.0, 104.0], [128.0, 240.0], [264.0, 184.0]], dtype=np.float32)

_NUNITS = 32
_TPU_ = _NT // _NUNITS
_BPB = 2
_NSTEP = _B // _BPB


def _sc_rows_body(pred_slabs, targ, rows_out, tv_vmem, sem):
    c = lax.axis_index("core")
    s = lax.axis_index("subcore")
    unit = c * 16 + s
    pltpu.sync_copy(targ, tv_vmem)
    lanes = lax.iota(jnp.int32, 16)
    scale = jnp.where(lanes >= 2, jnp.float32(_GW), jnp.float32(1.0))
    copies = []
    for r in range(_TPU_):
        j = unit * _TPU_ + r
        row = tv_vmem[j, :]
        scaled = row * scale
        ints = (scaled - lax.rem(scaled, 1.0)).astype(jnp.int32)
        bi = jnp.clip(ints[0], 0, _B - 1)
        gx = jnp.clip(ints[2], 0, _GW - 1)
        gy = jnp.clip(ints[3], 0, _GH - 1)
        cell = bi * (_A * _GH * _GW) + gy * _GW + gx
        for a in range(_A):
            g = (cell + a * (_GH * _GW)) // 8
            copies.append(pltpu.make_async_copy(
                pred_slabs.at[pl.ds(g, 1)],
                rows_out.at[pl.ds(a * _NT + j, 1)], sem))
    for cp in copies:
        cp.start()
    for cp in copies:
        cp.wait()


def _dense_kernel(pred_ref, out_ref, acc_ref):
    b = pl.program_id(0)
    nb = pl.num_programs(0)

    accs = [None] * 8
    idx = 0
    for i in range(_BPB):
        for a in range(_A):
            for g0 in range(0, _GH, 2):
                slab = pred_ref[i, a, g0:g0 + 2, :, :].reshape(128, _NCH)
                tr = jnp.transpose(slab)
                v = tr[4, :]
                sg = jax.nn.sigmoid(v)
                l1m = jnp.maximum(jnp.log(1.0 - sg), -100.0)
                k = idx % len(accs)
                accs[k] = l1m if accs[k] is None else accs[k] + l1m
                idx += 1
    tot = accs[0]
    for acc in accs[1:]:
        tot = tot + acc
    prev = jnp.where(b == 0, 0.0, acc_ref[0])
    acc_ref[0] = prev + jnp.sum(tot)

    @pl.when(b == nb - 1)
    def _():
        out_ref[0, 0] = acc_ref[0]


def _sparse_kernel(t_smem, t_vmem, slab_ref, dsum_ref, out_ref, rows_ref):
    for j in range(_NT):
        gxs = jnp.clip((t_smem[j, 2] * _GW).astype(jnp.int32), 0, _GW - 1)
        sub = lax.rem(gxs, 8)
        for a in range(_A):
            rows_ref[a, j, :] = slab_ref[a * _NT + j, sub, :]

    t = t_vmem[...]
    bv = jnp.clip(t[:, 0].astype(jnp.int32), 0, _B - 1)
    clsv = jnp.clip(t[:, 1].astype(jnp.int32), 0, _C - 1)
    x = t[:, 2] * _GW
    y = t[:, 3] * _GH
    w = t[:, 4] * _INPUT_DIM
    h = t[:, 5] * _INPUT_DIM
    valid = (x >= 0) & (y >= 0) & (x <= _GW - 1) & (y <= _GH - 1)
    gx = jnp.clip(x.astype(jnp.int32), 0, _GW - 1)
    gy = jnp.clip(y.astype(jnp.int32), 0, _GH - 1)

    inter0 = jnp.maximum(0.0, jnp.minimum(w, _SA[0, 0])) * jnp.maximum(0.0, jnp.minimum(h, _SA[0, 1]))
    inter1 = jnp.maximum(0.0, jnp.minimum(w, _SA[1, 0])) * jnp.maximum(0.0, jnp.minimum(h, _SA[1, 1]))
    inter2 = jnp.maximum(0.0, jnp.minimum(w, _SA[2, 0])) * jnp.maximum(0.0, jnp.minimum(h, _SA[2, 1]))
    wh = w * h
    iou0 = inter0 / (wh + _SA[0, 0] * _SA[0, 1] - inter0 + 1e-16)
    iou1 = inter1 / (wh + _SA[1, 0] * _SA[1, 1] - inter1 + 1e-16)
    iou2 = inter2 / (wh + _SA[2, 0] * _SA[2, 1] - inter2 + 1e-16)
    iv = jnp.where(iou1 > iou0, 1, 0)
    iv = jnp.where(iou2 > jnp.maximum(iou0, iou1), 2, iv)

    saw = jnp.where(iv == 0, _SA[0, 0], jnp.where(iv == 1, _SA[1, 0], _SA[2, 0]))
    sah = jnp.where(iv == 0, _SA[0, 1], jnp.where(iv == 1, _SA[1, 1], _SA[2, 1]))
    tx = x - gx.astype(jnp.float32)
    ty = y - gy.astype(jnp.float32)
    tw = jnp.log(w / saw + 1e-16)
    th = jnp.log(h / sah + 1e-16)

    cell = ((bv * _A + iv) * _GH + gy) * _GW + gx
    pid = cell * _C + clsv

    row_i = jax.lax.broadcasted_iota(jnp.int32, (_NT, _NT), 0)
    col_i = jax.lax.broadcasted_iota(jnp.int32, (_NT, _NT), 1)
    vcol = valid[None, :]
    same_cell = cell[:, None] == cell[None, :]
    overwritten = jnp.any(same_cell & (col_i > row_i) & vcol, axis=1)
    winner = valid & ~overwritten
    same_pair = pid[:, None] == pid[None, :]
    dup_pair = jnp.any(same_pair & (col_i < row_i) & vcol, axis=1)
    clsrep = valid & ~dup_pair

    P = jnp.where((iv == 0)[:, None], rows_ref[0],
                  jnp.where((iv == 1)[:, None], rows_ref[1], rows_ref[2]))

    px = jax.nn.sigmoid(P[:, 0])
    py = jax.nn.sigmoid(P[:, 1])
    coord = (px - tx) ** 2 + (py - ty) ** 2 + (P[:, 2] - tw) ** 2 + (P[:, 3] - th) ** 2
    so = jax.nn.sigmoid(P[:, 4])
    lobj = jnp.maximum(jnp.log(so), -100.0)
    l1mo = jnp.maximum(jnp.log(1.0 - so), -100.0)
    Sc = jax.nn.sigmoid(P[:, 5:])
    lc1 = jnp.maximum(jnp.log(Sc), -100.0)
    lc0 = jnp.maximum(jnp.log(1.0 - Sc), -100.0)
    base_cls = jnp.sum(lc0, axis=1)

    term = coord - lobj + _NO_OBJ_W * l1mo - _CLS_W * base_cls
    onehot = (jax.lax.broadcasted_iota(jnp.int32, (_NT, _C), 1) == clsv[:, None])
    corr = jnp.sum(jnp.where(onehot, lc1 - lc0, 0.0), axis=1)

    sparse = (jnp.sum(jnp.where(winner, term, 0.0))
              - _CLS_W * jnp.sum(jnp.where(clsrep, corr, 0.0)))
    out_ref[0, 0] = sparse - _NO_OBJ_W * dsum_ref[0, 0]


@jax.jit
def _yolo_loss(predictions, targets):
    pred = predictions.reshape(_B, _A, _GH, _GW, _NCH)
    pred_slabs = predictions.reshape(_NCELL // 8, 8, _NCH)
    targ = targets.reshape(_NT, 6)
    targ16 = jnp.pad(targ, ((0, 0), (0, 10)))

    sc_rows = pl.kernel(
        _sc_rows_body,
        out_type=jax.ShapeDtypeStruct((_A * _NT, 8, _NCH), jnp.float32),
        mesh=plsc.VectorSubcoreMesh(core_axis_name="core",
                                    subcore_axis_name="subcore"),
        scratch_types=[pltpu.VMEM((_NT, 16), jnp.float32),
                       pltpu.SemaphoreType.DMA],
    )
    slabs = sc_rows(pred_slabs, targ16)

    dsum = pl.pallas_call(
        _dense_kernel,
        grid=(_NSTEP,),
        in_specs=[
            pl.BlockSpec((_BPB, _A, _GH, _GW, _NCH), lambda b: (b, 0, 0, 0, 0)),
        ],
        out_specs=pl.BlockSpec(memory_space=pltpu.SMEM),
        out_shape=jax.ShapeDtypeStruct((1, 1), jnp.float32),
        scratch_shapes=[pltpu.SMEM((1,), jnp.float32)],
    )(pred)

    out = pl.pallas_call(
        _sparse_kernel,
        in_specs=[
            pl.BlockSpec(memory_space=pltpu.SMEM),
            pl.BlockSpec((_NT, 6), lambda: (0, 0)),
            pl.BlockSpec((_A * _NT, 8, _NCH), lambda: (0, 0, 0)),
            pl.BlockSpec(memory_space=pltpu.SMEM),
        ],
        out_specs=pl.BlockSpec(memory_space=pltpu.SMEM),
        out_shape=jax.ShapeDtypeStruct((1, 1), jnp.float32),
        scratch_shapes=[pltpu.VMEM((_A, _NT, _NCH), jnp.float32)],
    )(targ, targ, slabs, dsum)
    return out[0, 0]


def kernel(predictions, targets):
    return _yolo_loss(predictions, targets)


# speedup vs baseline: 1.4659x; 1.4659x over previous
import jax
import jax.numpy as jnp
import numpy as np
from jax.experimental import pallas as pl
from jax.experimental.pallas import tpu as pltpu

_B, _A, _GH, _GW = 16, 3, 64, 64
_C = 80
_NCH = 5 + _C
_NT = 64
_INPUT_DIM = 512.0
_NO_OBJ_W = 0.5
_CLS_W = 1.0
_SA = np.array([[80.0, 104.0], [128.0, 240.0], [264.0, 184.0]], dtype=np.float32)


_BPB = 2
_NSTEP = _B // _BPB
_CROWS = _BPB * _A * _GH * _GW // 128
_CHGY = 4


def _loss_kernel(t_smem, t_vmem, pred_ref, out_ref, acc_ref, rows_ref, flag_ref):
    b = pl.program_id(0)
    nb = pl.num_programs(0)

    @pl.when(b == 0)
    def _():
        for blk in range(_NSTEP):
            flag_ref[blk] = 0
        for j in range(_NT):
            tb = jnp.clip(t_smem[j, 0].astype(jnp.int32), 0, _B - 1)
            flag_ref[tb // _BPB] = 1

    accs = [None] * 8
    idx = 0
    for i in range(_BPB):
        for a in range(_A):
            for g0 in range(0, _GH, 2):
                slab = pred_ref[i, a, g0:g0 + 2, :, :].reshape(128, _NCH)
                tr = jnp.transpose(slab)
                v = tr[4, :]
                s = jax.nn.sigmoid(v)
                l1m = jnp.maximum(jnp.log(1.0 - s), -100.0)
                k = idx % len(accs)
                accs[k] = l1m if accs[k] is None else accs[k] + l1m
                idx += 1
    tot = accs[0]
    for acc in accs[1:]:
        tot = tot + acc
    prev = jnp.where(b == 0, 0.0, acc_ref[0])
    acc_ref[0] = prev + jnp.sum(tot)

    @pl.when(flag_ref[b] == 1)
    def _():
        for j in range(_NT):
            tb = jnp.clip(t_smem[j, 0].astype(jnp.int32), 0, _B - 1)
            hit = (tb // _BPB) == b
            bloc = jnp.where(hit, tb % _BPB, 0)
            x = t_smem[j, 2] * _GW
            y = t_smem[j, 3] * _GH
            gx = jnp.clip(x.astype(jnp.int32), 0, _GW - 1)
            gy = jnp.clip(y.astype(jnp.int32), 0, _GH - 1)
            for a in range(_A):
                val = pred_ref[bloc, a, gy, gx, :]
                rows_ref[a, j, :] = jnp.where(hit, val, rows_ref[a, j, :])

    @pl.when(b == nb - 1)
    def _():
        t = t_vmem[...]
        bv = jnp.clip(t[:, 0].astype(jnp.int32), 0, _B - 1)
        clsv = jnp.clip(t[:, 1].astype(jnp.int32), 0, _C - 1)
        x = t[:, 2] * _GW
        y = t[:, 3] * _GH
        w = t[:, 4] * _INPUT_DIM
        h = t[:, 5] * _INPUT_DIM
        valid = (x >= 0) & (y >= 0) & (x <= _GW - 1) & (y <= _GH - 1)
        gx = jnp.clip(x.astype(jnp.int32), 0, _GW - 1)
        gy = jnp.clip(y.astype(jnp.int32), 0, _GH - 1)

        inter0 = jnp.maximum(0.0, jnp.minimum(w, _SA[0, 0])) * jnp.maximum(0.0, jnp.minimum(h, _SA[0, 1]))
        inter1 = jnp.maximum(0.0, jnp.minimum(w, _SA[1, 0])) * jnp.maximum(0.0, jnp.minimum(h, _SA[1, 1]))
        inter2 = jnp.maximum(0.0, jnp.minimum(w, _SA[2, 0])) * jnp.maximum(0.0, jnp.minimum(h, _SA[2, 1]))
        wh = w * h
        iou0 = inter0 / (wh + _SA[0, 0] * _SA[0, 1] - inter0 + 1e-16)
        iou1 = inter1 / (wh + _SA[1, 0] * _SA[1, 1] - inter1 + 1e-16)
        iou2 = inter2 / (wh + _SA[2, 0] * _SA[2, 1] - inter2 + 1e-16)
        iv = jnp.where(iou1 > iou0, 1, 0)
        iv = jnp.where(iou2 > jnp.maximum(iou0, iou1), 2, iv)

        saw = jnp.where(iv == 0, _SA[0, 0], jnp.where(iv == 1, _SA[1, 0], _SA[2, 0]))
        sah = jnp.where(iv == 0, _SA[0, 1], jnp.where(iv == 1, _SA[1, 1], _SA[2, 1]))
        tx = x - gx.astype(jnp.float32)
        ty = y - gy.astype(jnp.float32)
        tw = jnp.log(w / saw + 1e-16)
        th = jnp.log(h / sah + 1e-16)

        cell = ((bv * _A + iv) * _GH + gy) * _GW + gx
        pid = cell * _C + clsv

        row_i = jax.lax.broadcasted_iota(jnp.int32, (_NT, _NT), 0)
        col_i = jax.lax.broadcasted_iota(jnp.int32, (_NT, _NT), 1)
        vcol = valid[None, :]
        same_cell = cell[:, None] == cell[None, :]
        overwritten = jnp.any(same_cell & (col_i > row_i) & vcol, axis=1)
        winner = valid & ~overwritten
        same_pair = pid[:, None] == pid[None, :]
        dup_pair = jnp.any(same_pair & (col_i < row_i) & vcol, axis=1)
        clsrep = valid & ~dup_pair

        P = jnp.where((iv == 0)[:, None], rows_ref[0],
                      jnp.where((iv == 1)[:, None], rows_ref[1], rows_ref[2]))

        px = jax.nn.sigmoid(P[:, 0])
        py = jax.nn.sigmoid(P[:, 1])
        coord = (px - tx) ** 2 + (py - ty) ** 2 + (P[:, 2] - tw) ** 2 + (P[:, 3] - th) ** 2
        so = jax.nn.sigmoid(P[:, 4])
        lobj = jnp.maximum(jnp.log(so), -100.0)
        l1mo = jnp.maximum(jnp.log(1.0 - so), -100.0)
        Sc = jax.nn.sigmoid(P[:, 5:])
        lc1 = jnp.maximum(jnp.log(Sc), -100.0)
        lc0 = jnp.maximum(jnp.log(1.0 - Sc), -100.0)
        base_cls = jnp.sum(lc0, axis=1)

        term = coord - lobj + _NO_OBJ_W * l1mo - _CLS_W * base_cls
        onehot = (jax.lax.broadcasted_iota(jnp.int32, (_NT, _C), 1) == clsv[:, None])
        corr = jnp.sum(jnp.where(onehot, lc1 - lc0, 0.0), axis=1)

        sparse = (jnp.sum(jnp.where(winner, term, 0.0))
                  - _CLS_W * jnp.sum(jnp.where(clsrep, corr, 0.0)))
        out_ref[0, 0] = sparse - _NO_OBJ_W * acc_ref[0]


def kernel(predictions, targets):
    pred = predictions.reshape(_B, _A, _GH, _GW, _NCH)
    targ = targets.reshape(_NT, 6)
    out = pl.pallas_call(
        _loss_kernel,
        grid=(_NSTEP,),
        in_specs=[
            pl.BlockSpec(memory_space=pltpu.SMEM),
            pl.BlockSpec((_NT, 6), lambda b: (0, 0)),
            pl.BlockSpec((_BPB, _A, _GH, _GW, _NCH), lambda b: (b, 0, 0, 0, 0)),
        ],
        out_specs=pl.BlockSpec(memory_space=pltpu.SMEM),
        out_shape=jax.ShapeDtypeStruct((1, 1), jnp.float32),
        scratch_shapes=[
            pltpu.SMEM((1,), jnp.float32),
            pltpu.VMEM((_A, _NT, _NCH), jnp.float32),
            pltpu.SMEM((_NSTEP,), jnp.int32),
        ],
    )(targ, targ, pred)
    return out[0, 0]
